# Initial kernel scaffold; baseline (speedup 1.0000x reference)
import jax
import jax.numpy as jnp
from jax.experimental import pallas as pl


def kernel(all_hidden_states, mask):
    raise NotImplementedError("write your pallas kernel here")


# speedup vs baseline: 847.4855x; 847.4855x over previous
import jax
import jax.numpy as jnp
from jax.experimental import pallas as pl
from jax.experimental.pallas import tpu as pltpu

_LS = 4
_L = 9
_WS = 2
_EPS = 1e-8
_SC = 256


def _window(k):
    left = list(range(k - _WS, k)) if k - _WS >= 0 else []
    right = list(range(k + 1, min(k + _WS + 1, _L)))
    return left + right + [k]


def _pairs():
    ps = set()
    for i in range(_L):
        ps.add((i, i))
    for k in range(_L):
        idx = _window(k)
        for a in range(len(idx)):
            for b in range(a + 1, len(idx)):
                i, j = idx[a], idx[b]
                ps.add((min(i, j), max(i, j)))
    return sorted(ps)


_PAIRS = _pairs()


def _body(a_ref, b_ref, c_ref, m_ref, out_ref, z_ref):
    c = pl.program_id(1)
    nc = pl.num_programs(1)
    D = a_ref.shape[-1]

    def lane_blk(i, cb):
        sl = slice(cb * 128, (cb + 1) * 128)
        if i < 4:
            return a_ref[i, 0, :, sl]
        if i < 8:
            return b_ref[i - 4, 0, :, sl]
        return c_ref[0, 0, :, sl]

    nblk = D // 128

    g = {}
    for (i, j) in _PAIRS:
        acc = lane_blk(i, 0) * lane_blk(j, 0)
        for cb in range(1, nblk):
            acc = acc + lane_blk(i, cb) * lane_blk(j, cb)
        t = jax.lax.transpose(acc, (1, 0))
        g[(i, j)] = jnp.sum(t, axis=0, keepdims=True)

    n = [jnp.sqrt(g[(i, i)]) for i in range(_L)]
    rn = [1.0 / n[i] for i in range(_L)]

    cs = [g[(l, l + 1)] / jnp.maximum(n[l] * n[l + 1], _EPS)
          for l in range(_L - 1)]
    cmean = sum(cs) * (1.0 / (_L - 1))
    var_raw = sum((x - cmean) ** 2 for x in cs) * (1.0 / (_L - 2))

    um = jnp.sum(m_ref[0].astype(jnp.float32), axis=-1, keepdims=True) - 1.0
    pos = (jax.lax.broadcasted_iota(jnp.int32, (1, _SC), 1)
           + c * _SC).astype(jnp.float32)
    var_m = jnp.where(pos < um, var_raw, 0.0)

    aligns, novs = [], []
    for k in range(_L):
        idx = _window(k)
        w = len(idx)

        def M(a, b, idx=idx):
            i0, j0 = idx[a], idx[b]
            return g[(min(i0, j0), max(i0, j0))]

        R = {}
        for a in range(w):
            s = M(a, a)
            for q in range(a):
                s = s - R[(q, a)] * R[(q, a)]
            inv = jax.lax.rsqrt(s)
            R[(a, a)] = s * inv
            for b2 in range(a + 1, w):
                t = M(a, b2)
                for q in range(a):
                    t = t - R[(q, a)] * R[(q, b2)]
                R[(a, b2)] = t * inv

        a_num = None
        for a in range(w - 1):
            t = None
            for j2 in range(a, w - 1):
                term = R[(a, j2)] * rn[idx[j2]]
                t = term if t is None else t + term
            mr = t * (1.0 / (w - 1))
            contrib = mr * R[(a, w - 1)]
            a_num = contrib if a_num is None else a_num + contrib
        r_pre = jnp.sqrt(sum(R[(a, w - 1)] ** 2 for a in range(w - 1)))
        aligns.append(r_pre / (a_num * (2.0 * w)))
        novs.append(R[(w - 1, w - 1)] * rn[k])

    inv_sa = 1.0 / sum(aligns)
    inv_sn = 1.0 / sum(novs)
    alphas = [al * inv_sa + nv * inv_sn for al, nv in zip(aligns, novs)]
    scale = var_m / sum(alphas)
    coef = [al * scale for al in alphas]

    ca = jnp.concatenate(coef[0:4], axis=1)
    cb4 = jnp.concatenate(coef[4:8], axis=1)
    fa = a_ref[:, 0].reshape(4 * _SC, D)
    fb = b_ref[:, 0].reshape(4 * _SC, D)
    fc = c_ref[0, 0]
    part = (jnp.dot(ca, fa, preferred_element_type=jnp.float32)
            + jnp.dot(cb4, fb, preferred_element_type=jnp.float32)
            + jnp.dot(coef[8], fc, preferred_element_type=jnp.float32))

    z_part = jnp.sum(var_m, axis=-1, keepdims=True)

    @pl.when(c == 0)
    def _():
        z_ref[...] = jnp.zeros_like(z_ref)
        out_ref[...] = jnp.zeros_like(out_ref)

    z_ref[...] = z_ref[...] + z_part
    out_ref[0] = out_ref[0] + part

    @pl.when(c == nc - 1)
    def _():
        out_ref[0] = out_ref[0] * (1.0 / z_ref[...])


def kernel(all_hidden_states, mask):
    NL, B, S, D = all_hidden_states.shape
    mask32 = mask.astype(jnp.int32).reshape(B, 1, S)
    nc = S // _SC
    x = all_hidden_states

    out = pl.pallas_call(
        _body,
        grid=(B, nc),
        in_specs=[
            pl.BlockSpec((4, 1, _SC, D), lambda b, c: (1, b, c, 0)),
            pl.BlockSpec((4, 1, _SC, D), lambda b, c: (2, b, c, 0)),
            pl.BlockSpec((1, 1, _SC, D), lambda b, c: (12, b, c, 0)),
            pl.BlockSpec((1, 1, S), lambda b, c: (b, 0, 0)),
        ],
        out_specs=pl.BlockSpec((1, 1, D), lambda b, c: (b, 0, 0)),
        out_shape=jax.ShapeDtypeStruct((B, 1, D), jnp.float32),
        scratch_shapes=[pltpu.VMEM((1, 1), jnp.float32)],
        compiler_params=pltpu.CompilerParams(
            dimension_semantics=("parallel", "arbitrary"),
        ),
    )(x, x, x, mask32)
    return out.reshape(B, D)


# speedup vs baseline: 877.4531x; 1.0354x over previous
import jax
import jax.numpy as jnp
from jax.experimental import pallas as pl
from jax.experimental.pallas import tpu as pltpu

_LS = 4
_L = 9
_WS = 2
_EPS = 1e-8
_SC = 512


def _window(k):
    left = list(range(k - _WS, k)) if k - _WS >= 0 else []
    right = list(range(k + 1, min(k + _WS + 1, _L)))
    return left + right + [k]


def _pairs():
    ps = set()
    for i in range(_L):
        ps.add((i, i))
    for k in range(_L):
        idx = _window(k)
        for a in range(len(idx)):
            for b in range(a + 1, len(idx)):
                i, j = idx[a], idx[b]
                ps.add((min(i, j), max(i, j)))
    return sorted(ps)


_PAIRS = _pairs()


def _body(a_ref, b_ref, c_ref, m_ref, out_ref, z_ref):
    c = pl.program_id(1)
    nc = pl.num_programs(1)
    D = a_ref.shape[-1]

    def lane_blk(i, cb, rsl):
        sl = slice(cb * 128, (cb + 1) * 128)
        if i < 4:
            return a_ref[i, 0, rsl, sl]
        if i < 8:
            return b_ref[i - 4, 0, rsl, sl]
        return c_ref[0, 0, rsl, sl]

    nblk = D // 128
    T = 128

    gch = {p: [] for p in _PAIRS}
    for t in range(_SC // T):
        rsl = slice(t * T, (t + 1) * T)
        for i in range(_L):
            partners = [j for j in range(i, min(i + 2 * _WS, _L - 1) + 1)]
            accs = [None] * len(partners)
            for cb in range(nblk):
                xi = lane_blk(i, cb, rsl)
                for pi, j in enumerate(partners):
                    xj = xi if j == i else lane_blk(j, cb, rsl)
                    p = xi * xj
                    accs[pi] = p if accs[pi] is None else accs[pi] + p
            for pi, j in enumerate(partners):
                tr = jax.lax.transpose(accs[pi], (1, 0))
                gch[(i, j)].append(jnp.sum(tr, axis=0, keepdims=True))
    g = {p: jnp.concatenate(gch[p], axis=1) for p in _PAIRS}

    n = [jnp.sqrt(g[(i, i)]) for i in range(_L)]
    rn = [1.0 / n[i] for i in range(_L)]

    cs = [g[(l, l + 1)] / jnp.maximum(n[l] * n[l + 1], _EPS)
          for l in range(_L - 1)]
    cmean = sum(cs) * (1.0 / (_L - 1))
    var_raw = sum((x - cmean) ** 2 for x in cs) * (1.0 / (_L - 2))

    um = jnp.sum(m_ref[0].astype(jnp.float32), axis=-1, keepdims=True) - 1.0
    pos = (jax.lax.broadcasted_iota(jnp.int32, (1, _SC), 1)
           + c * _SC).astype(jnp.float32)
    var_m = jnp.where(pos < um, var_raw, 0.0)

    aligns, novs = [], []
    for k in range(_L):
        idx = _window(k)
        w = len(idx)

        def M(a, b, idx=idx):
            i0, j0 = idx[a], idx[b]
            return g[(min(i0, j0), max(i0, j0))]

        R = {}
        for a in range(w):
            s = M(a, a)
            for q in range(a):
                s = s - R[(q, a)] * R[(q, a)]
            inv = jax.lax.rsqrt(s)
            R[(a, a)] = s * inv
            for b2 in range(a + 1, w):
                t = M(a, b2)
                for q in range(a):
                    t = t - R[(q, a)] * R[(q, b2)]
                R[(a, b2)] = t * inv

        a_num = None
        for a in range(w - 1):
            t = None
            for j2 in range(a, w - 1):
                term = R[(a, j2)] * rn[idx[j2]]
                t = term if t is None else t + term
            mr = t * (1.0 / (w - 1))
            contrib = mr * R[(a, w - 1)]
            a_num = contrib if a_num is None else a_num + contrib
        r_pre = jnp.sqrt(sum(R[(a, w - 1)] ** 2 for a in range(w - 1)))
        aligns.append(r_pre / (a_num * (2.0 * w)))
        novs.append(R[(w - 1, w - 1)] * rn[k])

    inv_sa = 1.0 / sum(aligns)
    inv_sn = 1.0 / sum(novs)
    alphas = [al * inv_sa + nv * inv_sn for al, nv in zip(aligns, novs)]
    scale = var_m / sum(alphas)
    coef = [al * scale for al in alphas]

    ca = jnp.concatenate(coef[0:4], axis=1)
    cb4 = jnp.concatenate(coef[4:8], axis=1)
    fa = a_ref[:, 0].reshape(4 * _SC, D)
    fb = b_ref[:, 0].reshape(4 * _SC, D)
    fc = c_ref[0, 0]
    part = (jnp.dot(ca, fa, preferred_element_type=jnp.float32)
            + jnp.dot(cb4, fb, preferred_element_type=jnp.float32)
            + jnp.dot(coef[8], fc, preferred_element_type=jnp.float32))

    z_part = jnp.sum(var_m, axis=-1, keepdims=True)

    @pl.when(c == 0)
    def _():
        z_ref[...] = jnp.zeros_like(z_ref)
        out_ref[...] = jnp.zeros_like(out_ref)

    z_ref[...] = z_ref[...] + z_part
    out_ref[0] = out_ref[0] + part

    @pl.when(c == nc - 1)
    def _():
        out_ref[0] = out_ref[0] * (1.0 / z_ref[...])


def kernel(all_hidden_states, mask):
    NL, B, S, D = all_hidden_states.shape
    mask32 = mask.astype(jnp.int32).reshape(B, 1, S)
    nc = S // _SC
    x = all_hidden_states

    out = pl.pallas_call(
        _body,
        grid=(B, nc),
        in_specs=[
            pl.BlockSpec((4, 1, _SC, D), lambda b, c: (1, b, c, 0)),
            pl.BlockSpec((4, 1, _SC, D), lambda b, c: (2, b, c, 0)),
            pl.BlockSpec((1, 1, _SC, D), lambda b, c: (12, b, c, 0)),
            pl.BlockSpec((1, 1, S), lambda b, c: (b, 0, 0)),
        ],
        out_specs=pl.BlockSpec((1, 1, D), lambda b, c: (b, 0, 0)),
        out_shape=jax.ShapeDtypeStruct((B, 1, D), jnp.float32),
        scratch_shapes=[pltpu.VMEM((1, 1), jnp.float32)],
        compiler_params=pltpu.CompilerParams(
            dimension_semantics=("parallel", "arbitrary"),
        ),
    )(x, x, x, mask32)
    return out.reshape(B, D)
